# Initial kernel scaffold; baseline (speedup 1.0000x reference)
import jax
import jax.numpy as jnp
from jax.experimental import pallas as pl


def kernel(act_prob, idx_del_prob, idx_add_prob, idx_arm_prob):
    raise NotImplementedError("write your pallas kernel here")


# speedup vs baseline: 1.5534x; 1.5534x over previous
import jax
import jax.numpy as jnp
from jax import lax
from jax.experimental import pallas as pl
from jax.experimental.pallas import tpu as pltpu

_INTERPRET = False

E = 65536
N = 4096
V = 8192
DROWS = E // V
ROWS = DROWS + N
BLK = 8


def _body(act_ref, d_ref, ad_ref, arm_ref, out_ref):
    i = pl.program_id(0)
    a0 = act_ref[0, 0]
    a1 = act_ref[0, 1]

    @pl.when(i == 0)
    def _():
        out_ref[...] = a0 * d_ref[...]

    @pl.when(i > 0)
    def _():
        out_ref[...] = (a1 * ad_ref[...]) * arm_ref[...]


def kernel(act_prob, idx_del_prob, idx_add_prob, idx_arm_prob):
    d2 = idx_del_prob.reshape(DROWS, V)
    grid = (ROWS // BLK,)
    out = pl.pallas_call(
        _body,
        grid=grid,
        in_specs=[
            pl.BlockSpec(memory_space=pltpu.SMEM),
            pl.BlockSpec((DROWS, V), lambda i: (0, 0)),
            pl.BlockSpec((BLK, 1), lambda i: (jnp.maximum(i - 1, 0), 0)),
            pl.BlockSpec((BLK, V), lambda i: (jnp.maximum(i - 1, 0), 0)),
        ],
        out_specs=pl.BlockSpec((BLK, V), lambda i: (i, 0)),
        out_shape=jax.ShapeDtypeStruct((ROWS, V), jnp.float32),
        interpret=_INTERPRET,
    )(act_prob, d2, idx_add_prob, idx_arm_prob)
    return out.reshape(-1)


# speedup vs baseline: 3.3957x; 2.1859x over previous
import jax
import jax.numpy as jnp
from jax import lax
from jax.experimental import pallas as pl
from jax.experimental.pallas import tpu as pltpu

_INTERPRET = False

E = 65536
N = 4096
V = 8192
DROWS = E // V
SLABS = N // DROWS + 1
B1 = 27


def _body(act_ref, d_ref, adp_ref, armp_ref, adm_ref, armm_ref, out_ref):
    k = pl.program_id(0)
    a0 = act_ref[0, 0]
    a1 = act_ref[0, 1]

    @pl.when(k == 0)
    def _():
        out_ref[0:1] = a0 * d_ref[...]

    @pl.when(k > 0)
    def _():
        out_ref[0:1] = (a1 * adp_ref[...]) * armp_ref[...]

    out_ref[1:B1] = (a1 * adm_ref[0:B1 - 1]) * armm_ref[0:B1 - 1]


def kernel(act_prob, idx_del_prob, idx_add_prob, idx_arm_prob):
    d3 = idx_del_prob.reshape(1, DROWS, V)
    arm3 = idx_arm_prob.reshape(SLABS - 1, DROWS, V)
    ad3 = idx_add_prob.reshape(SLABS - 1, DROWS, 1)
    grid = (SLABS // B1,)
    out = pl.pallas_call(
        _body,
        grid=grid,
        in_specs=[
            pl.BlockSpec(memory_space=pltpu.SMEM),
            pl.BlockSpec((1, DROWS, V), lambda k: (0, 0, 0)),
            pl.BlockSpec((1, DROWS, 1), lambda k: (jnp.maximum(B1 * k - 1, 0), 0, 0)),
            pl.BlockSpec((1, DROWS, V), lambda k: (jnp.maximum(B1 * k - 1, 0), 0, 0)),
            pl.BlockSpec((B1, DROWS, 1), lambda k: (k, 0, 0)),
            pl.BlockSpec((B1, DROWS, V), lambda k: (k, 0, 0)),
        ],
        out_specs=pl.BlockSpec((B1, DROWS, V), lambda k: (k, 0, 0)),
        out_shape=jax.ShapeDtypeStruct((SLABS, DROWS, V), jnp.float32),
        interpret=_INTERPRET,
    )(act_prob, d3, ad3, arm3, ad3, arm3)
    return out.reshape(-1)


# speedup vs baseline: 3.4097x; 1.0041x over previous
import jax
import jax.numpy as jnp
from jax import lax
from jax.experimental import pallas as pl
from jax.experimental.pallas import tpu as pltpu

_INTERPRET = False

E = 65536
N = 4096
V = 8192
DROWS = E // V
SLABS = N // DROWS + 1
B1 = 57


def _body(act_ref, d_ref, adp_ref, armp_ref, adm_ref, armm_ref, out_ref):
    k = pl.program_id(0)
    a0 = act_ref[0, 0]
    a1 = act_ref[0, 1]

    @pl.when(k == 0)
    def _():
        out_ref[0:1] = a0 * d_ref[...]

    @pl.when(k > 0)
    def _():
        out_ref[0:1] = (a1 * adp_ref[...]) * armp_ref[...]

    out_ref[1:B1] = (a1 * adm_ref[0:B1 - 1]) * armm_ref[0:B1 - 1]


def kernel(act_prob, idx_del_prob, idx_add_prob, idx_arm_prob):
    d3 = idx_del_prob.reshape(1, DROWS, V)
    arm3 = idx_arm_prob.reshape(SLABS - 1, DROWS, V)
    ad3 = idx_add_prob.reshape(SLABS - 1, DROWS, 1)
    grid = (SLABS // B1,)
    out = pl.pallas_call(
        _body,
        grid=grid,
        in_specs=[
            pl.BlockSpec(memory_space=pltpu.SMEM),
            pl.BlockSpec((1, DROWS, V), lambda k: (0, 0, 0)),
            pl.BlockSpec((1, DROWS, 1), lambda k: (jnp.maximum(B1 * k - 1, 0), 0, 0)),
            pl.BlockSpec((1, DROWS, V), lambda k: (jnp.maximum(B1 * k - 1, 0), 0, 0)),
            pl.BlockSpec((B1, DROWS, 1), lambda k: (k, 0, 0)),
            pl.BlockSpec((B1, DROWS, V), lambda k: (k, 0, 0)),
        ],
        out_specs=pl.BlockSpec((B1, DROWS, V), lambda k: (k, 0, 0)),
        out_shape=jax.ShapeDtypeStruct((SLABS, DROWS, V), jnp.float32),
        interpret=_INTERPRET,
    )(act_prob, d3, ad3, arm3, ad3, arm3)
    return out.reshape(-1)
